# baseline (speedup 1.0000x reference)
import jax
import jax.numpy as jnp
from jax import lax
from jax.experimental import pallas as pl
from jax.experimental.pallas import tpu as pltpu
from jax.experimental.pallas import tpu_sc as plsc

N_VOCAB = 100000
D_MODEL = 1024
BATCH = 4
SEQ = 4096
B_FLAT = BATCH * SEQ
LANES = 16
NSEG = D_MODEL // LANES

NC = 2
NS = 16
NW = NC * NS
POS_PW = SEQ // NW
P_CH = 8
NCH = POS_PW // P_CH
ROWS = BATCH * P_CH
NBUF = 2


def _embed_pe_kernel(idx_hbm, table_hbm, pe_hbm, out_hbm,
                     idx0, idx1, pe0, pe1, rows0, rows1,
                     in_sem0, in_sem1, out_sem0, out_sem1):
    c_id = lax.axis_index("c")
    s_id = lax.axis_index("s")
    wid = s_id * NC + c_id
    pos_base = wid * POS_PW

    idx_b = (idx0, idx1)
    pe_b = (pe0, pe1)
    rows_b = (rows0, rows1)
    in_sem = (in_sem0, in_sem1)
    out_sem = (out_sem0, out_sem1)

    def fetch(c, slot):
        pos0 = pos_base + c * P_CH
        for b in range(BATCH):
            pltpu.async_copy(
                idx_hbm.at[pl.ds(b * SEQ + pos0, P_CH)],
                idx_b[slot].at[pl.ds(b * P_CH, P_CH)],
                in_sem[slot],
            )
        pltpu.async_copy(pe_hbm.at[pl.ds(pos0, P_CH)], pe_b[slot], in_sem[slot])
        for b in range(BATCH):
            pltpu.make_async_copy(
                idx_hbm.at[pl.ds(b * SEQ + pos0, P_CH)],
                idx_b[slot].at[pl.ds(b * P_CH, P_CH)],
                in_sem[slot],
            ).wait()
        pltpu.async_copy(table_hbm.at[idx_b[slot]], rows_b[slot], in_sem[slot])

    def wait_in(c, slot):
        pos0 = pos_base + c * P_CH
        pltpu.make_async_copy(pe_hbm.at[pl.ds(pos0, P_CH)], pe_b[slot], in_sem[slot]).wait()
        pltpu.make_async_copy(table_hbm.at[idx_b[slot]], rows_b[slot], in_sem[slot]).wait()

    def store(c, slot):
        pos0 = pos_base + c * P_CH
        for b in range(BATCH):
            pltpu.async_copy(
                rows_b[slot].at[pl.ds(b * P_CH, P_CH)],
                out_hbm.at[pl.ds(b * SEQ + pos0, P_CH)],
                out_sem[slot],
            )

    def wait_store(c, slot):
        pos0 = pos_base + c * P_CH
        for b in range(BATCH):
            pltpu.make_async_copy(
                rows_b[slot].at[pl.ds(b * P_CH, P_CH)],
                out_hbm.at[pl.ds(b * SEQ + pos0, P_CH)],
                out_sem[slot],
            ).wait()

    def compute(slot):
        rows = rows_b[slot]
        pe_buf = pe_b[slot]

        def seg_body(j, carry2):
            col = j * LANES
            for p in range(P_CH):
                pv = pe_buf[p, pl.ds(col, LANES)]
                for b in range(BATCH):
                    r = b * P_CH + p
                    plsc.addupdate(rows.at[r, pl.ds(col, LANES)], pv)
            return carry2

        lax.fori_loop(0, NSEG, seg_body, 0)

    fetch(0, 0)

    def outer(i, carry):
        for b in range(NBUF):
            c = i * NBUF + b
            cur = b
            nxt = 1 - b

            @pl.when(c + 1 < NCH)
            def _():
                @pl.when(c >= 1)
                def _():
                    wait_store(c - 1, nxt)
                fetch(c + 1, nxt)

            wait_in(c, cur)
            compute(cur)
            store(c, cur)
        return carry

    lax.fori_loop(0, NCH // NBUF, outer, 0)
    wait_store(NCH - 2, 0)
    wait_store(NCH - 1, 1)


@jax.jit
def _run(idx, table, pe):
    mesh = plsc.VectorSubcoreMesh(core_axis_name="c", subcore_axis_name="s")
    k = pl.kernel(
        _embed_pe_kernel,
        mesh=mesh,
        out_type=jax.ShapeDtypeStruct((B_FLAT, D_MODEL), jnp.float32),
        scratch_types=[
            pltpu.VMEM((ROWS,), jnp.int32),
            pltpu.VMEM((ROWS,), jnp.int32),
            pltpu.VMEM((P_CH, D_MODEL), jnp.float32),
            pltpu.VMEM((P_CH, D_MODEL), jnp.float32),
            pltpu.VMEM((ROWS, D_MODEL), jnp.float32),
            pltpu.VMEM((ROWS, D_MODEL), jnp.float32),
            pltpu.SemaphoreType.DMA,
            pltpu.SemaphoreType.DMA,
            pltpu.SemaphoreType.DMA,
            pltpu.SemaphoreType.DMA,
        ],
    )
    return k(idx, table, pe)


def kernel(input, embed_table, pe):
    idx = input.astype(jnp.int32).reshape(B_FLAT)
    out = _run(idx, embed_table, pe)
    return out.reshape(BATCH, SEQ, D_MODEL)


# speedup vs baseline: 1.1699x; 1.1699x over previous
import jax
import jax.numpy as jnp
from jax import lax
from jax.experimental import pallas as pl
from jax.experimental.pallas import tpu as pltpu
from jax.experimental.pallas import tpu_sc as plsc

N_VOCAB = 100000
D_MODEL = 1024
BATCH = 4
SEQ = 4096
B_FLAT = BATCH * SEQ
LANES = 16
NSEG = D_MODEL // LANES

NC = 2
NS = 16
NW = NC * NS
POS_PW = SEQ // NW
P_CH = 8
NCH = POS_PW // P_CH
ROWS = BATCH * P_CH
NBUF = 2


def _embed_pe_kernel(idx_hbm, table_hbm, pe_hbm, out_hbm,
                     idx_all, pe0, pe1, rows0, rows1,
                     idx_sem, in_sem0, in_sem1, out_sem0, out_sem1):
    c_id = lax.axis_index("c")
    s_id = lax.axis_index("s")
    wid = s_id * NC + c_id
    pos_base = wid * POS_PW

    pe_b = (pe0, pe1)
    rows_b = (rows0, rows1)
    in_sem = (in_sem0, in_sem1)
    out_sem = (out_sem0, out_sem1)

    for b in range(BATCH):
        pltpu.async_copy(
            idx_hbm.at[pl.ds(b * SEQ + pos_base, POS_PW)],
            idx_all.at[b],
            idx_sem,
        )
    for b in range(BATCH):
        pltpu.make_async_copy(
            idx_hbm.at[pl.ds(b * SEQ + pos_base, POS_PW)],
            idx_all.at[b],
            idx_sem,
        ).wait()

    def fetch(c, slot):
        pos0 = pos_base + c * P_CH
        pltpu.async_copy(pe_hbm.at[pl.ds(pos0, P_CH)], pe_b[slot], in_sem[slot])
        for b in range(BATCH):
            pltpu.async_copy(
                table_hbm.at[idx_all.at[b, pl.ds(c * P_CH, P_CH)]],
                rows_b[slot].at[pl.ds(b * P_CH, P_CH)],
                in_sem[slot],
            )

    def wait_in(c, slot):
        pos0 = pos_base + c * P_CH
        pltpu.make_async_copy(pe_hbm.at[pl.ds(pos0, P_CH)], pe_b[slot], in_sem[slot]).wait()
        for b in range(BATCH):
            pltpu.make_async_copy(
                table_hbm.at[idx_all.at[b, pl.ds(c * P_CH, P_CH)]],
                rows_b[slot].at[pl.ds(b * P_CH, P_CH)],
                in_sem[slot],
            ).wait()

    def store(c, slot):
        pos0 = pos_base + c * P_CH
        for b in range(BATCH):
            pltpu.async_copy(
                rows_b[slot].at[pl.ds(b * P_CH, P_CH)],
                out_hbm.at[pl.ds(b * SEQ + pos0, P_CH)],
                out_sem[slot],
            )

    def wait_store(c, slot):
        pos0 = pos_base + c * P_CH
        for b in range(BATCH):
            pltpu.make_async_copy(
                rows_b[slot].at[pl.ds(b * P_CH, P_CH)],
                out_hbm.at[pl.ds(b * SEQ + pos0, P_CH)],
                out_sem[slot],
            ).wait()

    def compute(slot):
        rows = rows_b[slot]
        pe_buf = pe_b[slot]

        def seg_body(j, carry2):
            col = j * LANES
            for p in range(P_CH):
                pv = pe_buf[p, pl.ds(col, LANES)]
                for b in range(BATCH):
                    r = b * P_CH + p
                    rows[r, pl.ds(col, LANES)] = rows[r, pl.ds(col, LANES)] + pv
            return carry2

        lax.fori_loop(0, NSEG, seg_body, 0)

    fetch(0, 0)

    def outer(i, carry):
        for b in range(NBUF):
            c = i * NBUF + b
            cur = b
            nxt = 1 - b

            @pl.when(c + 1 < NCH)
            def _():
                @pl.when(c >= 1)
                def _():
                    wait_store(c - 1, nxt)
                fetch(c + 1, nxt)

            wait_in(c, cur)
            compute(cur)
            store(c, cur)
        return carry

    lax.fori_loop(0, NCH // NBUF, outer, 0)
    wait_store(NCH - 2, 0)
    wait_store(NCH - 1, 1)


@jax.jit
def _run(idx, table, pe):
    mesh = plsc.VectorSubcoreMesh(core_axis_name="c", subcore_axis_name="s")
    k = pl.kernel(
        _embed_pe_kernel,
        mesh=mesh,
        out_type=jax.ShapeDtypeStruct((B_FLAT, D_MODEL), jnp.float32),
        scratch_types=[
            pltpu.VMEM((BATCH, POS_PW), jnp.int32),
            pltpu.VMEM((P_CH, D_MODEL), jnp.float32),
            pltpu.VMEM((P_CH, D_MODEL), jnp.float32),
            pltpu.VMEM((ROWS, D_MODEL), jnp.float32),
            pltpu.VMEM((ROWS, D_MODEL), jnp.float32),
            pltpu.SemaphoreType.DMA,
            pltpu.SemaphoreType.DMA,
            pltpu.SemaphoreType.DMA,
            pltpu.SemaphoreType.DMA,
            pltpu.SemaphoreType.DMA,
        ],
    )
    return k(idx, table, pe)


def kernel(input, embed_table, pe):
    idx = input.astype(jnp.int32).reshape(B_FLAT)
    out = _run(idx, embed_table, pe)
    return out.reshape(BATCH, SEQ, D_MODEL)
